# baseline (speedup 1.0000x reference)
import functools

import jax
import jax.numpy as jnp
from jax import lax
from jax.experimental import pallas as pl
from jax.experimental.pallas import tpu as pltpu
from jax.experimental.pallas import tpu_sc as plsc

B = 2
S = 96
N = B * S * S * S
XZ = S * S
BI = 16

NC = 2
NS = 16
NW = NC * NS
PW = N // NW
C = 1024
NCH = PW // C
G = 128


def _prep_body(x0_ref, x1_ref, w_ref, q_ref, wy_ref, wx_ref, wz_ref):
    bi = pl.program_id(0)
    ip = pl.program_id(1)
    row0 = ip * BI

    def conv_step(a, accs):
        d0, d1, d2 = accs
        for ic in range(2):
            src = x0_ref if ic == 0 else x1_ref
            slab = src[0, pl.ds(row0 + a, BI), :, :].astype(jnp.float32)
            for bb in range(3):
                for cz in range(3):
                    sl = slab[:, bb:bb + S, cz:cz + S]
                    wbase = (a * 9 + bb * 3 + cz) * 6 + ic * 3
                    d0 = d0 + sl * w_ref[wbase]
                    d1 = d1 + sl * w_ref[wbase + 1]
                    d2 = d2 + sl * w_ref[wbase + 2]
        return (d0, d1, d2)

    zero = jnp.zeros((BI, S, S), jnp.float32)
    d0, d1, d2 = lax.fori_loop(0, 3, conv_step, (zero, zero, zero))

    ii = (lax.broadcasted_iota(jnp.int32, (BI, S, S), 0) + row0).astype(jnp.float32)
    jj = lax.broadcasted_iota(jnp.int32, (BI, S, S), 1).astype(jnp.float32)
    kk = lax.broadcasted_iota(jnp.int32, (BI, S, S), 2).astype(jnp.float32)
    x = jj + d0
    y = ii + d1
    z = kk + d2
    fx = jnp.floor(x).astype(jnp.int32)
    fy = jnp.floor(y).astype(jnp.int32)
    fz = jnp.floor(z).astype(jnp.int32)
    x0 = jnp.clip(fx, 0, S - 1)
    x1 = jnp.clip(fx + 1, 0, S - 1)
    y0 = jnp.clip(fy, 0, S - 1)
    y1 = jnp.clip(fy + 1, 0, S - 1)
    z0 = jnp.clip(fz, 0, S - 1)
    z1 = jnp.clip(fz + 1, 0, S - 1)
    bbase = bi * (S * S * S * 2)
    i000s = bbase + y0 * (XZ * 2) + x0 * (S * 2) + z0 * 2
    pk = (z1 - z0) + 2 * (x1 - x0) + 4 * (y1 - y0)
    q_ref[0] = (i000s << 3) | pk
    wy_ref[0] = y - y0.astype(jnp.float32)
    wx_ref[0] = x - x0.astype(jnp.float32)
    wz_ref[0] = z - z0.astype(jnp.float32)


def _prep(X, W_loc):
    Xp = jnp.pad(X.astype(jnp.bfloat16), ((0, 0), (1, 1), (1, 1), (1, 1), (0, 0)))
    X0 = Xp[..., 0]
    X1 = Xp[..., 1]
    wu = lax.bitcast_convert_type(W_loc, jnp.uint32)
    wu = (wu + 0x7FFF + ((wu >> 16) & 1)) & jnp.uint32(0xFFFF0000)
    Wf = lax.bitcast_convert_type(wu, jnp.float32).reshape(-1)
    vol = (B, S, S, S)
    blk = (1, BI, S, S)
    omap = lambda b, i: (b, i, 0, 0)
    in_specs = [
        pl.BlockSpec((1, S + 2, S + 2, S + 2), lambda b, i: (b, 0, 0, 0)),
        pl.BlockSpec((1, S + 2, S + 2, S + 2), lambda b, i: (b, 0, 0, 0)),
        pl.BlockSpec(memory_space=pltpu.SMEM),
    ]
    out_specs = [pl.BlockSpec(blk, omap)] * 4
    out_shape = ([jax.ShapeDtypeStruct(vol, jnp.int32)] +
                 [jax.ShapeDtypeStruct(vol, jnp.float32)] * 3)
    return pl.pallas_call(
        _prep_body,
        grid=(B, S // BI),
        in_specs=in_specs,
        out_specs=out_specs,
        out_shape=out_shape,
    )(X0, X1, Wf)


def _sc_body(q_h, wy_h, wx_h, wz_h, table_h, out_h,
             qv, wyv, wxv, wzv, idxv, valv, outv, semA, semB, semL):
    wid = lax.axis_index("s") * NC + lax.axis_index("c")
    start = wid * PW

    def loads(ci, slot):
        s = start + ci * C
        ds = [pltpu.make_async_copy(q_h.at[pl.ds(s, C)], qv.at[slot], semL),
              pltpu.make_async_copy(wy_h.at[pl.ds(s, C)], wyv.at[slot], semL),
              pltpu.make_async_copy(wx_h.at[pl.ds(s, C)], wxv.at[slot], semL),
              pltpu.make_async_copy(wz_h.at[pl.ds(s, C)], wzv.at[slot], semL)]
        for d in ds:
            d.start()
        for d in ds:
            d.wait()

    def p1(slot):
        def body(v, _):
            o = v * 16
            ds16 = pl.ds(o, 16)
            q = qv[slot, ds16]
            i0 = q >> 3
            dz = (q & 1) * 2
            dx = ((q >> 1) & 1) * (S * 2)
            dy = ((q >> 2) & 1) * (XZ * 2)
            iy = i0 + dy
            ix = i0 + dx
            iyx = iy + dx
            idxv[slot, 0, ds16] = i0
            idxv[slot, 1, ds16] = i0 + dz
            idxv[slot, 2, ds16] = ix
            idxv[slot, 3, ds16] = ix + dz
            idxv[slot, 4, ds16] = iy
            idxv[slot, 5, ds16] = iy + dz
            idxv[slot, 6, ds16] = iyx
            idxv[slot, 7, ds16] = iyx + dz
            return _
        lax.fori_loop(0, C // 16, body, None)

    def gathers(slot, sem):
        if G == C:
            return [pltpu.make_async_copy(
                table_h.at[idxv.at[slot, corner]],
                valv.at[slot, corner],
                sem)
                for corner in range(8)]
        return [pltpu.make_async_copy(
            table_h.at[idxv.at[slot, corner, pl.ds(j * G, G)]],
            valv.at[slot, corner, pl.ds(j * G, G)],
            sem)
            for corner in range(8) for j in range(C // G)]

    def fire(slot, sem):
        for d in gathers(slot, sem):
            d.start()

    def drain(slot, sem):
        for d in gathers(slot, sem):
            d.wait()

    def p2(slot):
        def body(v, _):
            o = v * 16
            ds16 = pl.ds(o, 16)
            p = qv[slot, ds16]
            wy1 = wyv[slot, ds16]
            wx1 = wxv[slot, ds16]
            wz1 = wzv[slot, ds16]
            wz0 = (p & 1).astype(jnp.float32) - wz1
            wx0 = ((p >> 1) & 1).astype(jnp.float32) - wx1
            wy0 = ((p >> 2) & 1).astype(jnp.float32) - wy1
            t00 = wy0 * wx0
            t01 = wy0 * wx1
            t10 = wy1 * wx0
            t11 = wy1 * wx1
            acc = (t00 * wz0) * valv[slot, 0, ds16]
            acc = acc + (t00 * wz1) * valv[slot, 1, ds16]
            acc = acc + (t01 * wz0) * valv[slot, 2, ds16]
            acc = acc + (t01 * wz1) * valv[slot, 3, ds16]
            acc = acc + (t10 * wz0) * valv[slot, 4, ds16]
            acc = acc + (t10 * wz1) * valv[slot, 5, ds16]
            acc = acc + (t11 * wz0) * valv[slot, 6, ds16]
            acc = acc + (t11 * wz1) * valv[slot, 7, ds16]
            outv[ds16] = acc
            return _
        lax.fori_loop(0, C // 16, body, None)

    def store(ci):
        pltpu.sync_copy(outv, out_h.at[pl.ds(start + ci * C, C)])

    loads(0, 0)
    p1(0)
    fire(0, semA)

    def body(h, _):
        c0 = 2 * h
        loads(c0 + 1, 1)
        p1(1)
        fire(1, semB)
        drain(0, semA)
        p2(0)
        store(c0)

        @pl.when(h < NCH // 2 - 1)
        def _prefetch():
            loads(c0 + 2, 0)
            p1(0)
            fire(0, semA)

        drain(1, semB)
        p2(1)
        store(c0 + 1)
        return _

    lax.fori_loop(0, NCH // 2, body, None)


def _sc_gather(q, wy, wx, wz, table):
    mesh = plsc.VectorSubcoreMesh(core_axis_name="c", subcore_axis_name="s")
    fn = pl.kernel(
        _sc_body,
        out_type=jax.ShapeDtypeStruct((N,), jnp.float32),
        mesh=mesh,
        scratch_types=[
            pltpu.VMEM((2, C), jnp.int32),
            pltpu.VMEM((2, C), jnp.float32),
            pltpu.VMEM((2, C), jnp.float32),
            pltpu.VMEM((2, C), jnp.float32),
            pltpu.VMEM((2, 8, C), jnp.int32),
            pltpu.VMEM((2, 8, C), jnp.float32),
            pltpu.VMEM((C,), jnp.float32),
            pltpu.SemaphoreType.DMA,
            pltpu.SemaphoreType.DMA,
            pltpu.SemaphoreType.DMA,
        ],
    )
    return fn(q, wy, wx, wz, table)


def kernel(X, W_loc):
    q, wy, wx, wz = _prep(X, W_loc)
    table = X.reshape(-1)
    flat = lambda a: a.reshape(-1)
    out = _sc_gather(flat(q), flat(wy), flat(wx), flat(wz), table)
    return out.reshape(B, S, S, S, 1)


# speedup vs baseline: 2.3915x; 2.3915x over previous
import functools

import jax
import jax.numpy as jnp
from jax import lax
from jax.experimental import pallas as pl
from jax.experimental.pallas import tpu as pltpu
from jax.experimental.pallas import tpu_sc as plsc

B = 2
S = 96
N = B * S * S * S
XZ = S * S
BI = 16

NC = 2
NS = 16
NW = NC * NS
PW = N // NW
C = 1024
NCH = PW // C
G = 128


def _prep_body(x0_ref, x1_ref, w_ref, q_ref, wy_ref, wx_ref, wz_ref):
    bi = pl.program_id(0)
    ip = pl.program_id(1)
    row0 = ip * BI

    def conv_step(a, accs):
        d0, d1, d2 = accs
        for ic in range(2):
            src = x0_ref if ic == 0 else x1_ref
            slab = src[0, pl.ds(row0 + a, BI), :, :].astype(jnp.float32)
            for bb in range(3):
                for cz in range(3):
                    sl = slab[:, bb:bb + S, cz:cz + S]
                    wbase = (a * 9 + bb * 3 + cz) * 6 + ic * 3
                    d0 = d0 + sl * w_ref[wbase]
                    d1 = d1 + sl * w_ref[wbase + 1]
                    d2 = d2 + sl * w_ref[wbase + 2]
        return (d0, d1, d2)

    zero = jnp.zeros((BI, S, S), jnp.float32)
    d0, d1, d2 = lax.fori_loop(0, 3, conv_step, (zero, zero, zero))

    ii = (lax.broadcasted_iota(jnp.int32, (BI, S, S), 0) + row0).astype(jnp.float32)
    jj = lax.broadcasted_iota(jnp.int32, (BI, S, S), 1).astype(jnp.float32)
    kk = lax.broadcasted_iota(jnp.int32, (BI, S, S), 2).astype(jnp.float32)
    x = jj + d0
    y = ii + d1
    z = kk + d2
    fx = jnp.floor(x).astype(jnp.int32)
    fy = jnp.floor(y).astype(jnp.int32)
    fz = jnp.floor(z).astype(jnp.int32)
    x0 = jnp.clip(fx, 0, S - 1)
    x1 = jnp.clip(fx + 1, 0, S - 1)
    y0 = jnp.clip(fy, 0, S - 1)
    y1 = jnp.clip(fy + 1, 0, S - 1)
    z0 = jnp.clip(fz, 0, S - 1)
    z1 = jnp.clip(fz + 1, 0, S - 1)
    bbase = bi * (S * S * S)
    i000 = bbase + y0 * XZ + x0 * S + z0
    pk = (z1 - z0) + 2 * (x1 - x0) + 4 * (y1 - y0)
    q_ref[0] = (i000 << 3) | pk
    wy_ref[0] = y - y0.astype(jnp.float32)
    wx_ref[0] = x - x0.astype(jnp.float32)
    wz_ref[0] = z - z0.astype(jnp.float32)


def _prep(X, W_loc):
    Xp = jnp.pad(X.astype(jnp.bfloat16), ((0, 0), (1, 1), (1, 1), (1, 1), (0, 0)))
    X0 = Xp[..., 0]
    X1 = Xp[..., 1]
    wu = lax.bitcast_convert_type(W_loc, jnp.uint32)
    wu = (wu + 0x7FFF + ((wu >> 16) & 1)) & jnp.uint32(0xFFFF0000)
    Wf = lax.bitcast_convert_type(wu, jnp.float32).reshape(-1)
    vol = (B, S, S, S)
    blk = (1, BI, S, S)
    omap = lambda b, i: (b, i, 0, 0)
    in_specs = [
        pl.BlockSpec((1, S + 2, S + 2, S + 2), lambda b, i: (b, 0, 0, 0)),
        pl.BlockSpec((1, S + 2, S + 2, S + 2), lambda b, i: (b, 0, 0, 0)),
        pl.BlockSpec(memory_space=pltpu.SMEM),
    ]
    out_specs = [pl.BlockSpec(blk, omap)] * 4
    out_shape = ([jax.ShapeDtypeStruct(vol, jnp.int32)] +
                 [jax.ShapeDtypeStruct(vol, jnp.float32)] * 3)
    return pl.pallas_call(
        _prep_body,
        grid=(B, S // BI),
        in_specs=in_specs,
        out_specs=out_specs,
        out_shape=out_shape,
    )(X0, X1, Wf)


def _sc_body(q_h, wy_h, wx_h, wz_h, table_h, out_h,
             qv, wyv, wxv, wzv, idxv, valv, outv, semA, semB, semL):
    wid = lax.axis_index("s") * NC + lax.axis_index("c")
    start = wid * PW

    def loads(ci, slot):
        s = start + ci * C
        ds = [pltpu.make_async_copy(q_h.at[pl.ds(s, C)], qv.at[slot], semL),
              pltpu.make_async_copy(wy_h.at[pl.ds(s, C)], wyv.at[slot], semL),
              pltpu.make_async_copy(wx_h.at[pl.ds(s, C)], wxv.at[slot], semL),
              pltpu.make_async_copy(wz_h.at[pl.ds(s, C)], wzv.at[slot], semL)]
        for d in ds:
            d.start()
        for d in ds:
            d.wait()

    def p1(slot):
        def body(v, _):
            o = v * 16
            ds16 = pl.ds(o, 16)
            q = qv[slot, ds16]
            i0 = q >> 3
            dz = q & 1
            dx = ((q >> 1) & 1) * S
            dy = ((q >> 2) & 1) * XZ
            iy = i0 + dy
            ix = i0 + dx
            iyx = iy + dx
            idxv[slot, 0, ds16] = i0
            idxv[slot, 1, ds16] = i0 + dz
            idxv[slot, 2, ds16] = ix
            idxv[slot, 3, ds16] = ix + dz
            idxv[slot, 4, ds16] = iy
            idxv[slot, 5, ds16] = iy + dz
            idxv[slot, 6, ds16] = iyx
            idxv[slot, 7, ds16] = iyx + dz
            return _
        lax.fori_loop(0, C // 16, body, None)

    def gathers(slot, sem):
        if G == C:
            return [pltpu.make_async_copy(
                table_h.at[idxv.at[slot, corner]],
                valv.at[slot, corner],
                sem)
                for corner in range(8)]
        return [pltpu.make_async_copy(
            table_h.at[idxv.at[slot, corner, pl.ds(j * G, G)]],
            valv.at[slot, corner, pl.ds(j * G, G)],
            sem)
            for corner in range(8) for j in range(C // G)]

    def fire(slot, sem):
        for d in gathers(slot, sem):
            d.start()

    def drain(slot, sem):
        for d in gathers(slot, sem):
            d.wait()

    def p2(slot):
        def body(v, _):
            o = v * 16
            ds16 = pl.ds(o, 16)
            p = qv[slot, ds16]
            wy1 = wyv[slot, ds16]
            wx1 = wxv[slot, ds16]
            wz1 = wzv[slot, ds16]
            wz0 = (p & 1).astype(jnp.float32) - wz1
            wx0 = ((p >> 1) & 1).astype(jnp.float32) - wx1
            wy0 = ((p >> 2) & 1).astype(jnp.float32) - wy1
            t00 = wy0 * wx0
            t01 = wy0 * wx1
            t10 = wy1 * wx0
            t11 = wy1 * wx1
            acc = (t00 * wz0) * valv[slot, 0, ds16]
            acc = acc + (t00 * wz1) * valv[slot, 1, ds16]
            acc = acc + (t01 * wz0) * valv[slot, 2, ds16]
            acc = acc + (t01 * wz1) * valv[slot, 3, ds16]
            acc = acc + (t10 * wz0) * valv[slot, 4, ds16]
            acc = acc + (t10 * wz1) * valv[slot, 5, ds16]
            acc = acc + (t11 * wz0) * valv[slot, 6, ds16]
            acc = acc + (t11 * wz1) * valv[slot, 7, ds16]
            outv[ds16] = acc
            return _
        lax.fori_loop(0, C // 16, body, None)

    def store(ci):
        pltpu.sync_copy(outv, out_h.at[pl.ds(start + ci * C, C)])

    loads(0, 0)
    p1(0)
    fire(0, semA)

    def body(h, _):
        c0 = 2 * h
        loads(c0 + 1, 1)
        p1(1)
        fire(1, semB)
        drain(0, semA)
        p2(0)
        store(c0)

        @pl.when(h < NCH // 2 - 1)
        def _prefetch():
            loads(c0 + 2, 0)
            p1(0)
            fire(0, semA)

        drain(1, semB)
        p2(1)
        store(c0 + 1)
        return _

    lax.fori_loop(0, NCH // 2, body, None)


def _sc_gather(q, wy, wx, wz, table):
    mesh = plsc.VectorSubcoreMesh(core_axis_name="c", subcore_axis_name="s")
    fn = pl.kernel(
        _sc_body,
        out_type=jax.ShapeDtypeStruct((N,), jnp.float32),
        mesh=mesh,
        scratch_types=[
            pltpu.VMEM((2, C), jnp.int32),
            pltpu.VMEM((2, C), jnp.float32),
            pltpu.VMEM((2, C), jnp.float32),
            pltpu.VMEM((2, C), jnp.float32),
            pltpu.VMEM((2, 8, C), jnp.int32),
            pltpu.VMEM((2, 8, C), jnp.float32),
            pltpu.VMEM((C,), jnp.float32),
            pltpu.SemaphoreType.DMA,
            pltpu.SemaphoreType.DMA,
            pltpu.SemaphoreType.DMA,
        ],
    )
    return fn(q, wy, wx, wz, table)


def kernel(X, W_loc):
    q, wy, wx, wz = _prep(X, W_loc)
    table = X[..., 0].reshape(-1)
    flat = lambda a: a.reshape(-1)
    out = _sc_gather(flat(q), flat(wy), flat(wx), flat(wz), table)
    return out.reshape(B, S, S, S, 1)
